# baseline (speedup 1.0000x reference)
import functools

import jax
import jax.numpy as jnp
from jax import lax
from jax.experimental import pallas as pl
from jax.experimental.pallas import tpu as pltpu
from jax.experimental.pallas import tpu_sc as plsc

D = 64
CHUNK = 128
G = 4


def _gather_body(n_groups, b_per_w, num_cores, table_hbm, idx_hbm, out_hbm,
                 idx_v, rows_v, sem_g0, sem_g1, sem_w0, sem_w1):
    sems_g = (sem_g0, sem_g1)
    sems_w = (sem_w0, sem_w1)
    gr = G * CHUNK
    wid = lax.axis_index("s") * num_cores + lax.axis_index("c")
    base = wid * b_per_w
    pltpu.sync_copy(idx_hbm.at[wid], idx_v)

    def issue_gathers(g, s):
        for b in range(G):
            pltpu.async_copy(
                table_hbm.at[idx_v.at[g * G + b]],
                rows_v.at[s, pl.ds(b * CHUNK, CHUNK)],
                sems_g[s])

    def wait_gathers(s):
        for b in range(G):
            pltpu.make_async_copy(
                table_hbm.at[idx_v.at[b]],
                rows_v.at[s, pl.ds(b * CHUNK, CHUNK)],
                sems_g[s]).wait()

    def wait_write(s):
        pltpu.make_async_copy(
            rows_v.at[s], out_hbm.at[pl.ds(base, gr)], sems_w[s]).wait()

    def round_(g, s, first, last):
        o = 1 - s
        wait_gathers(s)
        pltpu.async_copy(
            rows_v.at[s], out_hbm.at[pl.ds(base + g * gr, gr)], sems_w[s])
        if not last:
            if not first:
                wait_write(o)
            issue_gathers(g + 1, o)

    issue_gathers(0, 0)
    round_(0, 0, True, False)

    def pair(i, carry):
        round_(2 * i + 1, 1, False, False)
        round_(2 * i + 2, 0, False, False)
        return carry

    lax.fori_loop(0, (n_groups - 2) // 2, pair, 0)
    round_(n_groups - 1, 1, False, True)
    wait_write(0)
    wait_write(1)


def kernel(x, embed_table):
    B, H = x.shape
    total = B * H
    info = plsc.get_sparse_core_info()
    nw = info.num_cores * info.num_subcores
    b_per_w = total // nw
    n_chunks = b_per_w // CHUNK
    n_groups = n_chunks // G
    assert b_per_w * nw == total and n_groups * G * CHUNK == b_per_w
    assert n_groups % 2 == 0

    idx = x.reshape(nw, n_chunks, CHUNK).astype(jnp.int32)
    mesh = plsc.VectorSubcoreMesh(core_axis_name="c", subcore_axis_name="s")

    gather = functools.partial(
        pl.kernel,
        mesh=mesh,
        out_type=jax.ShapeDtypeStruct((total, D), jnp.float32),
        scratch_types=[
            pltpu.VMEM((n_chunks, CHUNK), jnp.int32),
            pltpu.VMEM((2, G * CHUNK, D), jnp.float32),
        ] + [pltpu.SemaphoreType.DMA] * 4,
        compiler_params=pltpu.CompilerParams(use_tc_tiling_on_sc=False),
    )(functools.partial(_gather_body, n_groups, b_per_w, info.num_cores))

    out = gather(embed_table, idx)
    return out.reshape(B, H, D)
